# baseline (speedup 1.0000x reference)
import functools

import numpy as np

import jax
import jax.numpy as jnp
from jax import lax
from jax.experimental import pallas as pl
from jax.experimental.pallas import tpu as pltpu
from jax.experimental.pallas import tpu_sc as plsc

_NC = 2
_NS = 16


def _sc_gather_bf16(word_table, token_indices):
    V, D = word_table.shape
    (B,) = token_indices.shape
    NW = _NC * _NS
    b_per_w = B // NW
    C = 32
    n_chunks = b_per_w // C

    mesh = plsc.VectorSubcoreMesh(core_axis_name="c", subcore_axis_name="s")

    @functools.partial(
        pl.kernel,
        mesh=mesh,
        out_type=jax.ShapeDtypeStruct((B, D // 2), jnp.int32),
        scratch_types=[
            pltpu.VMEM((b_per_w,), jnp.int32),
            pltpu.VMEM((C, D), jnp.int32),
            pltpu.VMEM((C, D), jnp.int32),
            pltpu.VMEM((C, D // 2), jnp.int32),
            pltpu.VMEM((C, D // 2), jnp.int32),
            pltpu.SemaphoreType.DMA,
            pltpu.SemaphoreType.DMA,
            pltpu.SemaphoreType.DMA,
            pltpu.SemaphoreType.DMA,
        ],
    )
    def gather_kernel(table_hbm, idx_hbm, out_hbm, idx_v, f0, f1, p0, p1,
                      g0, g1, s0, s1):
        wid = lax.axis_index("s") * _NC + lax.axis_index("c")
        base = wid * b_per_w
        pltpu.sync_copy(idx_hbm.at[pl.ds(base, b_per_w)], idx_v)

        fbufs = (f0, f1)
        pbufs = (p0, p1)
        gsems = (g0, g1)
        ssems = (s0, s1)

        def start_gather(i):
            pltpu.make_async_copy(
                table_hbm.at[idx_v.at[pl.ds(i * C, C)]], fbufs[i % 2],
                gsems[i % 2]).start()

        def wait_gather(i):
            pltpu.make_async_copy(
                table_hbm.at[idx_v.at[pl.ds(i * C, C)]], fbufs[i % 2],
                gsems[i % 2]).wait()

        def start_out(i):
            pltpu.make_async_copy(
                pbufs[i % 2], out_hbm.at[pl.ds(base + i * C, C)],
                ssems[i % 2]).start()

        def wait_out(i):
            pltpu.make_async_copy(
                pbufs[i % 2], out_hbm.at[pl.ds(base + i * C, C)],
                ssems[i % 2]).wait()

        def pack_chunk(i):
            fb = fbufs[i % 2]
            pb = pbufs[i % 2]

            def to_bf16_bits(ix):
                lsb = (ix >> 16) & jnp.int32(1)
                return ((ix + jnp.int32(0x7FFF) + lsb) >> 16) & jnp.int32(0xFFFF)

            def row(r, carry):
                for j in range(D // 32):
                    a = fb[r, pl.ds(j * 32, 16)]
                    b = fb[r, pl.ds(j * 32 + 16, 16)]
                    w = to_bf16_bits(a) | (to_bf16_bits(b) << 16)
                    pb[r, pl.ds(j * 16, 16)] = w
                return carry

            lax.fori_loop(0, C, row, 0)

        start_gather(0)
        for i in range(n_chunks):
            if i >= 2:
                wait_out(i - 2)
            if i + 1 < n_chunks:
                start_gather(i + 1)
            wait_gather(i)
            pack_chunk(i)
            start_out(i)
        wait_out(n_chunks - 2)
        wait_out(n_chunks - 1)

    return gather_kernel(word_table, token_indices)


def _pack_perm(D):
    p = np.empty(32, np.int32)
    for i in range(16):
        p[2 * i] = i
        p[2 * i + 1] = 16 + i
    return (np.arange(D, dtype=np.int32) // 32) * 32 + p[np.arange(D) % 32]


def _tc_matmul_chunk(prev, packed_k, tag3_k, w1wp, w1t, ttbf, b2, k, N, TILE):
    chunk, _ = packed_k.shape
    CD, D = w1wp.shape
    TAGS, TD = ttbf.shape
    tiles = chunk // TILE

    def body(*refs):
        if prev is None:
            tok_ref, tag_ref, w1_ref, w1t_ref, tt_ref, b_ref, out_ref = refs
        else:
            _, tok_ref, tag_ref, w1_ref, w1t_ref, tt_ref, b_ref, out_ref = refs
        tok = tok_ref[...]
        t = lax.dot_general(tt_ref[...], w1t_ref[...], (((1,), (1,)), ((), ())),
                            preferred_element_type=jnp.float32)
        tags = tag_ref[0, 0, :]
        oh = (tags[:, None]
              == lax.broadcasted_iota(jnp.int32, (TILE, TAGS), 1)
              ).astype(jnp.bfloat16)
        acc = lax.dot_general(tok, w1_ref[...], (((1,), (1,)), ((), ())),
                              preferred_element_type=jnp.float32)
        acc = acc + lax.dot_general(oh, t.astype(jnp.bfloat16),
                                    (((1,), (0,)), ((), ())),
                                    preferred_element_type=jnp.float32)
        out_ref[...] = acc + b_ref[...]

    in_specs = [
        pl.BlockSpec((TILE, D), lambda i: (i, 0)),
        pl.BlockSpec((1, 1, TILE), lambda i: (i, 0, 0)),
        pl.BlockSpec((CD, D), lambda i: (0, 0)),
        pl.BlockSpec((CD, TD), lambda i: (0, 0)),
        pl.BlockSpec((TAGS, TD), lambda i: (0, 0)),
        pl.BlockSpec((1, CD), lambda i: (0, 0)),
    ]
    args = [packed_k, tag3_k, w1wp, w1t, ttbf, b2]
    aliases = {}
    if prev is not None:
        in_specs = [pl.BlockSpec(memory_space=pl.ANY)] + in_specs
        args = [prev] + args
        aliases = {0: 0}

    return pl.pallas_call(
        body,
        grid=(tiles,),
        in_specs=in_specs,
        out_specs=pl.BlockSpec((TILE, CD), lambda i: (k * tiles + i, 0)),
        out_shape=jax.ShapeDtypeStruct((N, CD), jnp.float32),
        input_output_aliases=aliases,
    )(*args)


def kernel(token_indices, tag_indices, word_table, tag_table, W1, b1):
    tok = token_indices.astype(jnp.int32)
    tags = tag_indices.astype(jnp.int32)
    (N,) = tok.shape
    V, D = word_table.shape
    CD = W1.shape[0]
    K = 4
    TILE = 2048
    chunk = N // K

    perm = jnp.asarray(_pack_perm(D))
    w1wp = jnp.take(W1[:, :D], perm, axis=1).astype(jnp.bfloat16)
    w1t = W1[:, D:].astype(jnp.bfloat16)
    ttbf = tag_table.astype(jnp.bfloat16)
    b2 = b1.reshape(1, CD)

    wt_i32 = lax.bitcast_convert_type(word_table, jnp.int32)
    packed = [
        lax.bitcast_convert_type(
            _sc_gather_bf16(wt_i32,
                            lax.slice(tok, (k * chunk,), ((k + 1) * chunk,))),
            jnp.bfloat16).reshape(chunk, D)
        for k in range(K)
    ]
    out = None
    for k in range(K):
        tag3_k = lax.slice(tags, (k * chunk,), ((k + 1) * chunk,)).reshape(
            chunk // TILE, 1, TILE)
        out = _tc_matmul_chunk(out, packed[k], tag3_k, w1wp, w1t, ttbf, b2,
                               k, N, TILE)
    return out


# speedup vs baseline: 6.0883x; 6.0883x over previous
import functools

import numpy as np

import jax
import jax.numpy as jnp
from jax import lax
from jax.experimental import pallas as pl
from jax.experimental.pallas import tpu as pltpu
from jax.experimental.pallas import tpu_sc as plsc

_NC = 2
_NS = 16


def _sc_gather_bf16(word_table, token_indices):
    V, D = word_table.shape
    (B,) = token_indices.shape
    NW = _NC * _NS
    b_per_w = B // NW
    C = 32
    n_chunks = b_per_w // C

    mesh = plsc.VectorSubcoreMesh(core_axis_name="c", subcore_axis_name="s")

    @functools.partial(
        pl.kernel,
        mesh=mesh,
        out_type=jax.ShapeDtypeStruct((B, D), jnp.float32),
        scratch_types=[
            pltpu.VMEM((b_per_w,), jnp.int32),
            pltpu.VMEM((C, D), jnp.float32),
            pltpu.VMEM((C, D), jnp.float32),
            pltpu.SemaphoreType.DMA,
            pltpu.SemaphoreType.DMA,
            pltpu.SemaphoreType.DMA,
            pltpu.SemaphoreType.DMA,
        ],
    )
    def gather_kernel(table_hbm, idx_hbm, out_hbm, idx_v, f0, f1,
                      g0, g1, s0, s1):
        wid = lax.axis_index("s") * _NC + lax.axis_index("c")
        base = wid * b_per_w
        pltpu.sync_copy(idx_hbm.at[pl.ds(base, b_per_w)], idx_v)

        fbufs = (f0, f1)
        gsems = (g0, g1)
        ssems = (s0, s1)

        def start_gather(i):
            pltpu.make_async_copy(
                table_hbm.at[idx_v.at[pl.ds(i * C, C)]], fbufs[i % 2],
                gsems[i % 2]).start()

        def wait_gather(i):
            pltpu.make_async_copy(
                table_hbm.at[idx_v.at[pl.ds(i * C, C)]], fbufs[i % 2],
                gsems[i % 2]).wait()

        def start_out(i):
            pltpu.make_async_copy(
                fbufs[i % 2], out_hbm.at[pl.ds(base + i * C, C)],
                ssems[i % 2]).start()

        def wait_out(i):
            pltpu.make_async_copy(
                fbufs[i % 2], out_hbm.at[pl.ds(base + i * C, C)],
                ssems[i % 2]).wait()

        start_gather(0)
        for i in range(n_chunks):
            if i + 1 < n_chunks:
                if i >= 1:
                    wait_out(i - 1)
                start_gather(i + 1)
            wait_gather(i)
            start_out(i)
        wait_out(n_chunks - 2)
        wait_out(n_chunks - 1)

    return gather_kernel(word_table, token_indices)


def _pack_perm(D):
    p = np.empty(32, np.int32)
    for i in range(16):
        p[2 * i] = i
        p[2 * i + 1] = 16 + i
    return (np.arange(D, dtype=np.int32) // 32) * 32 + p[np.arange(D) % 32]


def _tc_matmul_chunk(prev, packed_k, tag3_k, w1wp, w1t, ttbf, b2, k, N, TILE):
    chunk, _ = packed_k.shape
    CD, D = w1wp.shape
    TAGS, TD = ttbf.shape
    tiles = chunk // TILE

    def body(*refs):
        if prev is None:
            tok_ref, tag_ref, w1_ref, w1t_ref, tt_ref, b_ref, out_ref = refs
        else:
            _, tok_ref, tag_ref, w1_ref, w1t_ref, tt_ref, b_ref, out_ref = refs
        tok = tok_ref[...].astype(jnp.bfloat16)
        t = lax.dot_general(tt_ref[...], w1t_ref[...], (((1,), (1,)), ((), ())),
                            preferred_element_type=jnp.float32)
        tags = tag_ref[0, 0, :]
        oh = (tags[:, None]
              == lax.broadcasted_iota(jnp.int32, (TILE, TAGS), 1)
              ).astype(jnp.bfloat16)
        acc = lax.dot_general(tok, w1_ref[...], (((1,), (1,)), ((), ())),
                              preferred_element_type=jnp.float32)
        acc = acc + lax.dot_general(oh, t.astype(jnp.bfloat16),
                                    (((1,), (0,)), ((), ())),
                                    preferred_element_type=jnp.float32)
        out_ref[...] = acc + b_ref[...]

    in_specs = [
        pl.BlockSpec((TILE, D), lambda i: (i, 0)),
        pl.BlockSpec((1, 1, TILE), lambda i: (i, 0, 0)),
        pl.BlockSpec((CD, D), lambda i: (0, 0)),
        pl.BlockSpec((CD, TD), lambda i: (0, 0)),
        pl.BlockSpec((TAGS, TD), lambda i: (0, 0)),
        pl.BlockSpec((1, CD), lambda i: (0, 0)),
    ]
    args = [packed_k, tag3_k, w1wp, w1t, ttbf, b2]
    aliases = {}
    if prev is not None:
        in_specs = [pl.BlockSpec(memory_space=pl.ANY)] + in_specs
        args = [prev] + args
        aliases = {0: 0}

    return pl.pallas_call(
        body,
        grid=(tiles,),
        in_specs=in_specs,
        out_specs=pl.BlockSpec((TILE, CD), lambda i: (k * tiles + i, 0)),
        out_shape=jax.ShapeDtypeStruct((N, CD), jnp.float32),
        input_output_aliases=aliases,
    )(*args)


def kernel(token_indices, tag_indices, word_table, tag_table, W1, b1):
    tok = token_indices.astype(jnp.int32)
    tags = tag_indices.astype(jnp.int32)
    (N,) = tok.shape
    V, D = word_table.shape
    CD = W1.shape[0]
    K = 4
    TILE = 2048
    chunk = N // K

    w1wp = W1[:, :D].astype(jnp.bfloat16)
    w1t = W1[:, D:].astype(jnp.bfloat16)
    ttbf = tag_table.astype(jnp.bfloat16)
    b2 = b1.reshape(1, CD)

    packed = [
        _sc_gather_bf16(word_table,
                        lax.slice(tok, (k * chunk,), ((k + 1) * chunk,)))
        for k in range(K)
    ]
    out = None
    for k in range(K):
        tag3_k = lax.slice(tags, (k * chunk,), ((k + 1) * chunk,)).reshape(
            chunk // TILE, 1, TILE)
        out = _tc_matmul_chunk(out, packed[k], tag3_k, w1wp, w1t, ttbf, b2,
                               k, N, TILE)
    return out
